# baseline (speedup 1.0000x reference)
import functools

import jax
import jax.numpy as jnp
from jax.experimental import pallas as pl


_TX = 512


def _mask_hi16(v):
    u = jax.lax.bitcast_convert_type(v, jnp.uint32)
    return jax.lax.bitcast_convert_type(u & jnp.uint32(0xFFFF0000), jnp.float32)


def _chunks4(v):
    c1 = _mask_hi16(v)
    r = v - c1
    c2 = _mask_hi16(r)
    r = r - c2
    c3 = _mask_hi16(r)
    r = r - c3
    c4 = _mask_hi16(r)
    return [c1, c2, c3, c4]


def _chamfer_kernel(xa_ref, yt_ref, y2_ref, out_ref, *, n, m):
    yt = yt_ref[0]
    y2r = y2_ref[0]

    def body(i, carry):
        row_sum, col_min = carry
        xs = xa_ref[0, pl.ds(i * _TX, _TX), :]
        mm = jax.lax.dot_general(
            xs, yt, (((1,), (0,)), ((), ())),
            preferred_element_type=jnp.float32)
        dist = mm + y2r
        row_sum = row_sum + jnp.sum(jnp.min(dist, axis=1))
        col_min = jnp.minimum(col_min, jnp.min(dist, axis=0, keepdims=True))
        return row_sum, col_min

    init = (jnp.float32(0.0), jnp.full((1, m), jnp.inf, jnp.float32))
    row_sum, col_min = jax.lax.fori_loop(0, n // _TX, body, init)
    dist_xy = row_sum / n
    dist_yx = jnp.sum(col_min) / m
    out_ref[0] = jnp.maximum(dist_xy, dist_yx).reshape(1, 1)


def kernel(x, y):
    b, n, _ = x.shape
    m = y.shape[1]
    ones = jnp.ones((b, n, 1), jnp.float32)
    x2 = jnp.sum(x * x, axis=2, keepdims=True)
    y2 = jnp.sum(y * y, axis=2, keepdims=True)
    xc = _chunks4(x2)
    zeros = jnp.zeros((b, n, 1), jnp.float32)
    xa = jnp.concatenate([-2.0 * x] + xc + [zeros], axis=2)
    ya = jnp.concatenate([y] + [ones] * 4 + [zeros], axis=2)
    yt = jnp.swapaxes(ya, 1, 2)
    y2r = jnp.swapaxes(y2, 1, 2)

    per_item = pl.pallas_call(
        functools.partial(_chamfer_kernel, n=n, m=m),
        grid=(b,),
        in_specs=[
            pl.BlockSpec((1, n, 8), lambda i: (i, 0, 0)),
            pl.BlockSpec((1, 8, m), lambda i: (i, 0, 0)),
            pl.BlockSpec((1, 1, m), lambda i: (i, 0, 0)),
        ],
        out_specs=pl.BlockSpec((1, 1, 1), lambda i: (i, 0, 0)),
        out_shape=jax.ShapeDtypeStruct((b, 1, 1), jnp.float32),
    )(xa, yt, y2r)
    return jnp.mean(per_item)


# speedup vs baseline: 1.0051x; 1.0051x over previous
import functools

import jax
import jax.numpy as jnp
from jax.experimental import pallas as pl


_TX = 512


def _mask_hi16(v):
    u = jax.lax.bitcast_convert_type(v, jnp.uint32)
    return jax.lax.bitcast_convert_type(u & jnp.uint32(0xFFFF0000), jnp.float32)


def _chunks4(v):
    c1 = _mask_hi16(v)
    r = v - c1
    c2 = _mask_hi16(r)
    r = r - c2
    c3 = _mask_hi16(r)
    r = r - c3
    c4 = _mask_hi16(r)
    return [c1, c2, c3, c4]


def _chamfer_kernel(xa_ref, yt_ref, out_ref, *, n, m):
    yt = yt_ref[0]
    y2r = yt[7:8, :]

    def body(i, carry):
        row_sum, col_min = carry
        xs = xa_ref[0, pl.ds(i * _TX, _TX), :]
        mm = jax.lax.dot_general(
            xs, yt, (((1,), (0,)), ((), ())),
            preferred_element_type=jnp.float32)
        dist = mm + y2r
        row_sum = row_sum + jnp.sum(jnp.min(dist, axis=1))
        col_min = jnp.minimum(col_min, jnp.min(dist, axis=0, keepdims=True))
        return row_sum, col_min

    init = (jnp.float32(0.0), jnp.full((1, m), jnp.inf, jnp.float32))
    row_sum, col_min = jax.lax.fori_loop(0, n // _TX, body, init)
    dist_xy = row_sum / n
    dist_yx = jnp.sum(col_min) / m
    out_ref[0] = jnp.maximum(dist_xy, dist_yx).reshape(1, 1)


def kernel(x, y):
    b, n, _ = x.shape
    m = y.shape[1]
    ones = jnp.ones((b, n, 1), jnp.float32)
    x2 = jnp.sum(x * x, axis=2, keepdims=True)
    y2 = jnp.sum(y * y, axis=2, keepdims=True)
    xc = _chunks4(x2)
    zeros = jnp.zeros((b, n, 1), jnp.float32)
    xa = jnp.concatenate([-2.0 * x] + xc + [zeros], axis=2)
    ya = jnp.concatenate([y] + [ones] * 4 + [y2], axis=2)
    yt = jnp.swapaxes(ya, 1, 2)

    per_item = pl.pallas_call(
        functools.partial(_chamfer_kernel, n=n, m=m),
        grid=(b,),
        in_specs=[
            pl.BlockSpec((1, n, 8), lambda i: (i, 0, 0)),
            pl.BlockSpec((1, 8, m), lambda i: (i, 0, 0)),
        ],
        out_specs=pl.BlockSpec((1, 1, 1), lambda i: (i, 0, 0)),
        out_shape=jax.ShapeDtypeStruct((b, 1, 1), jnp.float32),
    )(xa, yt)
    return jnp.mean(per_item)


# speedup vs baseline: 2.2416x; 2.2303x over previous
import functools

import jax
import jax.numpy as jnp
from jax.experimental import pallas as pl
from jax.experimental.pallas import tpu as pltpu


_TX = 512


def _mask_hi16(v):
    u = jax.lax.bitcast_convert_type(v, jnp.uint32)
    return jax.lax.bitcast_convert_type(u & jnp.uint32(0xFFFF0000), jnp.float32)


def _chunks4(v):
    c1 = _mask_hi16(v)
    r = v - c1
    c2 = _mask_hi16(r)
    r = r - c2
    c3 = _mask_hi16(r)
    r = r - c3
    c4 = _mask_hi16(r)
    return [c1, c2, c3, c4]


def _augment(ct, norm_first):
    n = ct.shape[1]
    sq = ct * ct
    norm = (sq[0:1] + sq[1:2]) + sq[2:3]
    chunks = _chunks4(norm)
    ones = jnp.ones((4, n), jnp.float32)
    zeros = jnp.zeros((5, n), jnp.float32)
    if norm_first:
        parts = [-2.0 * ct] + chunks + [ones, zeros]
    else:
        parts = [ct, ones] + chunks + [zeros]
    return jnp.concatenate(parts, axis=0)


def _chamfer_kernel(xt_ref, yt_ref, out_ref, xa_ref, *, n, m):
    xa_ref[...] = _augment(xt_ref[0], True)
    ya = _augment(yt_ref[0], False)

    def body(i, carry):
        row_sum, col_min = carry
        xs = xa_ref[:, pl.ds(i * _TX, _TX)]
        dist = jax.lax.dot_general(
            xs, ya, (((0,), (0,)), ((), ())),
            preferred_element_type=jnp.float32)
        row_sum = row_sum + jnp.sum(jnp.min(dist, axis=1))
        col_min = jnp.minimum(col_min, jnp.min(dist, axis=0, keepdims=True))
        return row_sum, col_min

    init = (jnp.float32(0.0), jnp.full((1, m), jnp.inf, jnp.float32))
    row_sum, col_min = jax.lax.fori_loop(0, n // _TX, body, init)
    dist_xy = row_sum / n
    dist_yx = jnp.sum(col_min) / m
    out_ref[0] = jnp.maximum(dist_xy, dist_yx).reshape(1, 1)


def kernel(x, y):
    b, n, _ = x.shape
    m = y.shape[1]
    xt = jnp.swapaxes(x, 1, 2)
    yt = jnp.swapaxes(y, 1, 2)

    per_item = pl.pallas_call(
        functools.partial(_chamfer_kernel, n=n, m=m),
        grid=(b,),
        in_specs=[
            pl.BlockSpec((1, 3, n), lambda i: (i, 0, 0)),
            pl.BlockSpec((1, 3, m), lambda i: (i, 0, 0)),
        ],
        out_specs=pl.BlockSpec((1, 1, 1), lambda i: (i, 0, 0)),
        out_shape=jax.ShapeDtypeStruct((b, 1, 1), jnp.float32),
        scratch_shapes=[pltpu.VMEM((16, n), jnp.float32)],
    )(xt, yt)
    return jnp.mean(per_item)


# speedup vs baseline: 2.5373x; 1.1319x over previous
import functools

import jax
import jax.numpy as jnp
from jax.experimental import pallas as pl
from jax.experimental.pallas import tpu as pltpu


_TX = 1024


def _mask_hi16(v):
    u = jax.lax.bitcast_convert_type(v, jnp.uint32)
    return jax.lax.bitcast_convert_type(u & jnp.uint32(0xFFFF0000), jnp.float32)


def _chunks4(v):
    c1 = _mask_hi16(v)
    r = v - c1
    c2 = _mask_hi16(r)
    r = r - c2
    c3 = _mask_hi16(r)
    r = r - c3
    c4 = _mask_hi16(r)
    return [c1, c2, c3, c4]


def _augment(ct, norm_first):
    n = ct.shape[1]
    sq = ct * ct
    norm = (sq[0:1] + sq[1:2]) + sq[2:3]
    chunks = _chunks4(norm)
    ones = jnp.ones((4, n), jnp.float32)
    zeros = jnp.zeros((5, n), jnp.float32)
    if norm_first:
        parts = [-2.0 * ct] + chunks + [ones, zeros]
    else:
        parts = [ct, ones] + chunks + [zeros]
    return jnp.concatenate(parts, axis=0)


def _chamfer_kernel(xt_ref, yt_ref, out_ref, xa_ref, *, n, m):
    xa_ref[...] = _augment(xt_ref[0], True)
    ya = _augment(yt_ref[0], False)

    def body(i, carry):
        row_sum, col_min = carry
        xs = xa_ref[:, pl.ds(i * _TX, _TX)]
        dist = jax.lax.dot_general(
            xs, ya, (((0,), (0,)), ((), ())),
            preferred_element_type=jnp.float32)
        row_sum = row_sum + jnp.sum(jnp.min(dist, axis=1))
        col_min = jnp.minimum(col_min, jnp.min(dist, axis=0, keepdims=True))
        return row_sum, col_min

    init = (jnp.float32(0.0), jnp.full((1, m), jnp.inf, jnp.float32))
    row_sum, col_min = jax.lax.fori_loop(0, n // _TX, body, init)
    dist_xy = row_sum / n
    dist_yx = jnp.sum(col_min) / m
    out_ref[0] = jnp.maximum(dist_xy, dist_yx).reshape(1, 1)


def kernel(x, y):
    b, n, _ = x.shape
    m = y.shape[1]
    xt = jnp.swapaxes(x, 1, 2)
    yt = jnp.swapaxes(y, 1, 2)

    per_item = pl.pallas_call(
        functools.partial(_chamfer_kernel, n=n, m=m),
        grid=(b,),
        in_specs=[
            pl.BlockSpec((1, 3, n), lambda i: (i, 0, 0)),
            pl.BlockSpec((1, 3, m), lambda i: (i, 0, 0)),
        ],
        out_specs=pl.BlockSpec((1, 1, 1), lambda i: (i, 0, 0)),
        out_shape=jax.ShapeDtypeStruct((b, 1, 1), jnp.float32),
        scratch_shapes=[pltpu.VMEM((16, n), jnp.float32)],
    )(xt, yt)
    return jnp.mean(per_item)


# speedup vs baseline: 2.7177x; 1.0711x over previous
import functools

import jax
import jax.numpy as jnp
from jax.experimental import pallas as pl
from jax.experimental.pallas import tpu as pltpu


_TX = 2048


def _mask_hi16(v):
    u = jax.lax.bitcast_convert_type(v, jnp.uint32)
    return jax.lax.bitcast_convert_type(u & jnp.uint32(0xFFFF0000), jnp.float32)


def _chunks4(v):
    c1 = _mask_hi16(v)
    r = v - c1
    c2 = _mask_hi16(r)
    r = r - c2
    c3 = _mask_hi16(r)
    r = r - c3
    c4 = _mask_hi16(r)
    return [c1, c2, c3, c4]


def _augment(ct, norm_first):
    n = ct.shape[1]
    sq = ct * ct
    norm = (sq[0:1] + sq[1:2]) + sq[2:3]
    chunks = _chunks4(norm)
    ones = jnp.ones((4, n), jnp.float32)
    zeros = jnp.zeros((5, n), jnp.float32)
    if norm_first:
        parts = [-2.0 * ct] + chunks + [ones, zeros]
    else:
        parts = [ct, ones] + chunks + [zeros]
    return jnp.concatenate(parts, axis=0)


def _chamfer_kernel(xt_ref, yt_ref, out_ref, xa_ref, *, n, m):
    xa_ref[...] = _augment(xt_ref[0], True)
    ya = _augment(yt_ref[0], False)

    def body(i, carry):
        row_sum, col_min = carry
        xs = xa_ref[:, pl.ds(i * _TX, _TX)]
        dist = jax.lax.dot_general(
            xs, ya, (((0,), (0,)), ((), ())),
            preferred_element_type=jnp.float32)
        row_sum = row_sum + jnp.sum(jnp.min(dist, axis=1))
        col_min = jnp.minimum(col_min, jnp.min(dist, axis=0, keepdims=True))
        return row_sum, col_min

    init = (jnp.float32(0.0), jnp.full((1, m), jnp.inf, jnp.float32))
    row_sum, col_min = jax.lax.fori_loop(0, n // _TX, body, init)
    dist_xy = row_sum / n
    dist_yx = jnp.sum(col_min) / m
    out_ref[0] = jnp.maximum(dist_xy, dist_yx).reshape(1, 1)


def kernel(x, y):
    b, n, _ = x.shape
    m = y.shape[1]
    xt = jnp.swapaxes(x, 1, 2)
    yt = jnp.swapaxes(y, 1, 2)

    per_item = pl.pallas_call(
        functools.partial(_chamfer_kernel, n=n, m=m),
        grid=(b,),
        in_specs=[
            pl.BlockSpec((1, 3, n), lambda i: (i, 0, 0)),
            pl.BlockSpec((1, 3, m), lambda i: (i, 0, 0)),
        ],
        out_specs=pl.BlockSpec((1, 1, 1), lambda i: (i, 0, 0)),
        out_shape=jax.ShapeDtypeStruct((b, 1, 1), jnp.float32),
        scratch_shapes=[pltpu.VMEM((16, n), jnp.float32)],
    )(xt, yt)
    return jnp.mean(per_item)
